# baseline (speedup 1.0000x reference)
import functools

import jax
import jax.numpy as jnp
from jax import lax
from jax.experimental import pallas as pl
from jax.experimental.pallas import tpu as pltpu
from jax.experimental.pallas import tpu_sc as plsc

NC = 2
NS = 16
L = 16
NW = NC * NS
CH = 80
NB = 8

_GDN = lax.GatherDimensionNumbers(
    offset_dims=(), collapsed_slice_dims=(0,), start_index_map=(0,))


def _lane_gather(v, idx):
    return lax.gather(v, idx[:, None], _GDN, (1,),
                      mode=lax.GatherScatterMode.PROMISE_IN_BOUNDS)


def _sc_edge_pass(table, packed, n_nodes, mode):
    E = packed.shape[0] * CH
    assert E % (NW * CH) == 0
    n_pad = ((n_nodes + NS * 128 - 1) // (NS * 128)) * (NS * 128)
    ept = E // NW
    nchunk = ept // CH
    assert nchunk >= NB and (nchunk - 2) % NB == 0
    rpt = n_pad // NS
    ZB = 128
    assert rpt % ZB == 0

    mesh = plsc.VectorSubcoreMesh(core_axis_name="c", subcore_axis_name="s")

    @functools.partial(
        pl.kernel,
        out_type=jax.ShapeDtypeStruct((NC, n_pad, L), jnp.float32),
        mesh=mesh,
        scratch_types=[
            pltpu.VMEM((NB * 3, CH), jnp.int32),
            pltpu.VMEM((NB, CH, L), jnp.float32),
            pltpu.VMEM((NB, CH, L), jnp.float32),
            pltpu.VMEM((ZB, L), jnp.float32),
            pltpu.VMEM_SHARED((n_pad, L), jnp.float32),
            pltpu.SemaphoreType.DMA((NB,)),
            pltpu.SemaphoreType.DMA((NB,)),
            pltpu.SemaphoreType.DMA((NB,)),
        ],
        compiler_params=pltpu.CompilerParams(use_tc_tiling_on_sc=False),
    )
    def kfn(table_h, pk_h, out_h, pkv, rows, outr,
            zb, acc, semI, semG, semS):
        cid = lax.axis_index("c")
        sid = lax.axis_index("s")
        wid = cid * NS + sid
        base_row = sid * rpt

        lane = lax.iota(jnp.int32, L)
        if mode == 1:
            maskf = jnp.where((lane >= 5) & (lane < 10), 1.0, 0.0)
        else:
            maskf = jnp.where(lane < 4, 1.0, 0.0)

        def zrow(i, _):
            zb[i, :] = jnp.zeros((L,), jnp.float32)
            return 0
        lax.fori_loop(0, ZB, zrow, 0)

        def zcp(k, _):
            pltpu.sync_copy(zb, acc.at[pl.ds(base_row + k * ZB, ZB)])
            return 0
        lax.fori_loop(0, rpt // ZB, zcp, 0)
        plsc.subcore_barrier()

        cbase = wid * nchunk

        def issue_idx(c, b):
            pltpu.async_copy(pk_h.at[cbase + c],
                             pkv.at[pl.ds(b * 3, 3)], semI.at[b])

        def wait_idx(b):
            pltpu.make_async_copy(pk_h.at[0], pkv.at[pl.ds(b * 3, 3)],
                                  semI.at[b]).wait()

        def issue_gather(b):
            pltpu.async_copy(table_h.at[pkv.at[b * 3]], rows.at[b],
                             semG.at[b])

        def wait_gather(b):
            pltpu.make_async_copy(
                table_h.at[pkv.at[b * 3]], rows.at[b], semG.at[b]).wait()

        def compute(b):
            for g in range(CH // L):
                uraw = lax.bitcast_convert_type(
                    pkv[b * 3 + 2, pl.ds(g * L, L)], jnp.float32)
                u16 = jnp.clip(uraw, 0.0, 1.0)
                if mode == 1:
                    a16 = u16 - 1.0
                else:
                    a16 = 1.0 - 2.0 * u16
                for i in range(L):
                    e = g * L + i
                    idx = jnp.full((L,), i, jnp.int32)
                    av = _lane_gather(a16, idx)
                    if mode == 1:
                        scale = maskf * av + 1.0
                    else:
                        bv = _lane_gather(u16, idx)
                        scale = maskf * av + bv
                    outr[b, e, :] = rows[b, e, :] * scale

        def issue_scatter(b):
            pltpu.async_copy(outr.at[b], acc.at[pkv.at[b * 3 + 1]],
                             semS.at[b], add=True)

        def wait_scatter(b):
            pltpu.make_async_copy(outr.at[b], acc.at[pkv.at[b * 3 + 1]],
                                  semS.at[b]).wait()

        issue_idx(0, 0)
        issue_idx(1, 1)
        issue_idx(2, 2)
        wait_idx(0)
        issue_gather(0)
        wait_idx(1)
        issue_gather(1)

        def main_body(cc, _):
            for p in range(NB):
                c = cc * NB + p
                b = p
                bg = (p + 2) % NB
                bi = (p + 3) % NB
                bs = (p + 6) % NB

                @pl.when(c >= 2)
                def _():
                    wait_scatter(bs)

                @pl.when(c + 3 <= nchunk - 1)
                def _():
                    issue_idx(c + 3, bi)
                wait_idx(bg)
                issue_gather(bg)
                wait_gather(b)
                compute(b)
                issue_scatter(b)
            return 0
        lax.fori_loop(0, (nchunk - 2) // NB, main_body, 0)

        wait_scatter(6)
        wait_gather(0)
        compute(0)
        issue_scatter(0)
        wait_scatter(7)
        wait_gather(1)
        compute(1)
        issue_scatter(1)
        wait_scatter(0)
        wait_scatter(1)
        plsc.subcore_barrier()

        pltpu.sync_copy(acc.at[pl.ds(base_row, rpt)],
                        out_h.at[cid, pl.ds(base_row, rpt)])

    return kfn(table, packed)


def _tc_node1(part, xdup, W10, W11, root1, b1, W20, W21):
    n = xdup.shape[0]
    BN = 2000
    assert n % BN == 0

    def body(p_ref, x_ref, w10, w11, r1, bb1, w20, w21,
             h_ref, z_ref, inv_ref):
        acc = p_ref[0] + p_ref[1]
        T = acc[:, 0:5]
        S = acc[:, 5:10]
        cnt = acc[:, 10:11]
        inv = 1.0 / jnp.maximum(cnt, 1.0)
        agg = (jnp.dot(T - S, w10[...], preferred_element_type=jnp.float32)
               + jnp.dot(S, w11[...], preferred_element_type=jnp.float32))
        agg = agg * inv
        x = x_ref[:, 0:5]
        h = agg + jnp.dot(x, r1[...], preferred_element_type=jnp.float32) \
            + bb1[...]
        h = jnp.where(h > 0, h, jnp.exp(jnp.minimum(h, 0.0)) - 1.0)
        h_ref[...] = h
        y0 = jnp.dot(h, w20[...], preferred_element_type=jnp.float32)
        y1 = jnp.dot(h, w21[...], preferred_element_type=jnp.float32)
        z_ref[...] = jnp.concatenate(
            [y0, y1, jnp.zeros((BN, 8), jnp.float32)], axis=1)
        inv_ref[...] = inv

    big = pl.BlockSpec((BN, L), lambda i: (i, 0))
    return pl.pallas_call(
        body,
        grid=(n // BN,),
        in_specs=[
            pl.BlockSpec((NC, BN, L), lambda i: (0, i, 0)),
            big,
            pl.BlockSpec((5, 16), lambda i: (0, 0)),
            pl.BlockSpec((5, 16), lambda i: (0, 0)),
            pl.BlockSpec((5, 16), lambda i: (0, 0)),
            pl.BlockSpec((1, 16), lambda i: (0, 0)),
            pl.BlockSpec((16, 4), lambda i: (0, 0)),
            pl.BlockSpec((16, 4), lambda i: (0, 0)),
        ],
        out_specs=[big, big, pl.BlockSpec((BN, 1), lambda i: (i, 0))],
        out_shape=[
            jax.ShapeDtypeStruct((n, L), jnp.float32),
            jax.ShapeDtypeStruct((n, L), jnp.float32),
            jax.ShapeDtypeStruct((n, 1), jnp.float32),
        ],
    )(part, xdup, W10, W11, root1, b1, W20, W21)


def _tc_node2(part, h, inv, root2, b2):
    n = h.shape[0]
    BN = 2000
    assert n % BN == 0

    def body(p_ref, h_ref, inv_ref, r2, bb2, o_ref):
        acc = p_ref[0] + p_ref[1]
        agg = (acc[:, 0:4] + acc[:, 4:8]) * inv_ref[...]
        o = agg + jnp.dot(h_ref[...], r2[...],
                          preferred_element_type=jnp.float32) + bb2[...]
        m = jnp.max(o, axis=1, keepdims=True)
        s = o - m
        lse = jnp.log(jnp.sum(jnp.exp(s), axis=1, keepdims=True))
        o_ref[...] = s - lse

    big = pl.BlockSpec((BN, L), lambda i: (i, 0))
    return pl.pallas_call(
        body,
        grid=(n // BN,),
        in_specs=[
            pl.BlockSpec((NC, BN, L), lambda i: (0, i, 0)),
            big,
            pl.BlockSpec((BN, 1), lambda i: (i, 0)),
            pl.BlockSpec((16, 4), lambda i: (0, 0)),
            pl.BlockSpec((1, 4), lambda i: (0, 0)),
        ],
        out_specs=pl.BlockSpec((BN, 4), lambda i: (i, 0)),
        out_shape=jax.ShapeDtypeStruct((n, 4), jnp.float32),
    )(part, h, inv, root2, b2)


def kernel(node_feature, edge_index, edge_feature, W1, root1, bias1,
           W2, root2, bias2):
    n = node_feature.shape[0]
    E = edge_index.shape[1]

    ei = jnp.transpose(edge_index.reshape(2, E // CH, CH), (1, 0, 2))
    ub = jax.lax.bitcast_convert_type(
        edge_feature.reshape(1, E // CH, CH), jnp.int32).reshape(
            E // CH, 1, CH)
    packed = jnp.concatenate([ei, ub], axis=1)

    xdup = jnp.concatenate(
        [node_feature, node_feature,
         jnp.ones((n, 1), jnp.float32),
         jnp.zeros((n, L - 11), jnp.float32)], axis=1)

    part1 = _sc_edge_pass(xdup, packed, n, mode=1)
    h, z, inv = _tc_node1(part1, xdup,
                          W1[0], W1[1], root1,
                          bias1.reshape(1, 16), W2[0], W2[1])
    part2 = _sc_edge_pass(z, packed, n, mode=2)
    return _tc_node2(part2, h, inv, root2, bias2.reshape(1, 4))


# speedup vs baseline: 1.0617x; 1.0617x over previous
import functools

import jax
import jax.numpy as jnp
from jax import lax
from jax.experimental import pallas as pl
from jax.experimental.pallas import tpu as pltpu
from jax.experimental.pallas import tpu_sc as plsc

NC = 2
NS = 16
L = 16
NW = NC * NS
CH = 80
NB = 8

_GDN = lax.GatherDimensionNumbers(
    offset_dims=(), collapsed_slice_dims=(0,), start_index_map=(0,))


def _lane_gather(v, idx):
    return lax.gather(v, idx[:, None], _GDN, (1,),
                      mode=lax.GatherScatterMode.PROMISE_IN_BOUNDS)


def _sc_edge_pass(table, packed, n_nodes, mode):
    E = packed.shape[1]
    assert E % (NW * CH) == 0
    n_pad = ((n_nodes + NS * 128 - 1) // (NS * 128)) * (NS * 128)
    ept = E // NW
    nchunk = ept // CH
    assert nchunk >= NB and (nchunk - 2) % NB == 0
    rpt = n_pad // NS
    ZB = 128
    assert rpt % ZB == 0

    mesh = plsc.VectorSubcoreMesh(core_axis_name="c", subcore_axis_name="s")

    @functools.partial(
        pl.kernel,
        out_type=jax.ShapeDtypeStruct((NC, n_pad, L), jnp.float32),
        mesh=mesh,
        scratch_types=[
            pltpu.VMEM((NB * 3, CH), jnp.int32),
            pltpu.VMEM((NB, CH, L), jnp.float32),
            pltpu.VMEM((NB, CH, L), jnp.float32),
            pltpu.VMEM((ZB, L), jnp.float32),
            pltpu.VMEM_SHARED((n_pad, L), jnp.float32),
            pltpu.SemaphoreType.DMA((NB,)),
            pltpu.SemaphoreType.DMA((NB,)),
            pltpu.SemaphoreType.DMA((NB,)),
        ],
        compiler_params=pltpu.CompilerParams(use_tc_tiling_on_sc=False),
    )
    def kfn(table_h, pk_h, out_h, pkv, rows, outr,
            zb, acc, semI, semG, semS):
        cid = lax.axis_index("c")
        sid = lax.axis_index("s")
        wid = cid * NS + sid
        base_row = sid * rpt

        lane = lax.iota(jnp.int32, L)
        if mode == 1:
            maskf = jnp.where((lane >= 5) & (lane < 10), 1.0, 0.0)
        else:
            maskf = jnp.where(lane < 4, 1.0, 0.0)

        def zrow(i, _):
            zb[i, :] = jnp.zeros((L,), jnp.float32)
            return 0
        lax.fori_loop(0, ZB, zrow, 0)

        def zcp(k, _):
            pltpu.sync_copy(zb, acc.at[pl.ds(base_row + k * ZB, ZB)])
            return 0
        lax.fori_loop(0, rpt // ZB, zcp, 0)
        plsc.subcore_barrier()

        ebase = wid * ept

        def issue_idx(c, b):
            o = ebase + c * CH
            pltpu.async_copy(pk_h.at[:, pl.ds(o, CH)],
                             pkv.at[pl.ds(b * 3, 3)], semI.at[b])

        def wait_idx(b):
            pltpu.make_async_copy(pk_h.at[:, pl.ds(0, CH)],
                                  pkv.at[pl.ds(b * 3, 3)],
                                  semI.at[b]).wait()

        def issue_gather(b):
            pltpu.async_copy(table_h.at[pkv.at[b * 3]], rows.at[b],
                             semG.at[b])

        def wait_gather(b):
            pltpu.make_async_copy(
                table_h.at[pkv.at[b * 3]], rows.at[b], semG.at[b]).wait()

        def compute(b):
            for g in range(CH // L):
                uraw = lax.bitcast_convert_type(
                    pkv[b * 3 + 2, pl.ds(g * L, L)], jnp.float32)
                u16 = jnp.clip(uraw, 0.0, 1.0)
                if mode == 1:
                    a16 = u16 - 1.0
                else:
                    a16 = 1.0 - 2.0 * u16
                for i in range(L):
                    e = g * L + i
                    idx = jnp.full((L,), i, jnp.int32)
                    av = _lane_gather(a16, idx)
                    if mode == 1:
                        scale = maskf * av + 1.0
                    else:
                        bv = _lane_gather(u16, idx)
                        scale = maskf * av + bv
                    outr[b, e, :] = rows[b, e, :] * scale

        def issue_scatter(b):
            pltpu.async_copy(outr.at[b], acc.at[pkv.at[b * 3 + 1]],
                             semS.at[b], add=True)

        def wait_scatter(b):
            pltpu.make_async_copy(outr.at[b], acc.at[pkv.at[b * 3 + 1]],
                                  semS.at[b]).wait()

        issue_idx(0, 0)
        issue_idx(1, 1)
        issue_idx(2, 2)
        wait_idx(0)
        issue_gather(0)
        wait_idx(1)
        issue_gather(1)

        def main_body(cc, _):
            for p in range(NB):
                c = cc * NB + p
                b = p
                bg = (p + 2) % NB
                bi = (p + 3) % NB
                bs = (p + 6) % NB

                @pl.when(c >= 2)
                def _():
                    wait_scatter(bs)

                @pl.when(c + 3 <= nchunk - 1)
                def _():
                    issue_idx(c + 3, bi)
                wait_idx(bg)
                issue_gather(bg)
                wait_gather(b)
                compute(b)
                issue_scatter(b)
            return 0
        lax.fori_loop(0, (nchunk - 2) // NB, main_body, 0)

        wait_scatter(6)
        wait_gather(0)
        compute(0)
        issue_scatter(0)
        wait_scatter(7)
        wait_gather(1)
        compute(1)
        issue_scatter(1)
        wait_scatter(0)
        wait_scatter(1)
        plsc.subcore_barrier()

        pltpu.sync_copy(acc.at[pl.ds(base_row, rpt)],
                        out_h.at[cid, pl.ds(base_row, rpt)])

    return kfn(table, packed)


def _tc_node1(part, xdup, W10, W11, root1, b1, W20, W21):
    n = xdup.shape[0]
    BN = 2000
    assert n % BN == 0

    def body(p_ref, x_ref, w10, w11, r1, bb1, w20, w21,
             h_ref, z_ref, inv_ref):
        acc = p_ref[0] + p_ref[1]
        T = acc[:, 0:5]
        S = acc[:, 5:10]
        cnt = acc[:, 10:11]
        inv = 1.0 / jnp.maximum(cnt, 1.0)
        agg = (jnp.dot(T - S, w10[...], preferred_element_type=jnp.float32)
               + jnp.dot(S, w11[...], preferred_element_type=jnp.float32))
        agg = agg * inv
        x = x_ref[:, 0:5]
        h = agg + jnp.dot(x, r1[...], preferred_element_type=jnp.float32) \
            + bb1[...]
        h = jnp.where(h > 0, h, jnp.exp(jnp.minimum(h, 0.0)) - 1.0)
        h_ref[...] = h
        y0 = jnp.dot(h, w20[...], preferred_element_type=jnp.float32)
        y1 = jnp.dot(h, w21[...], preferred_element_type=jnp.float32)
        z_ref[...] = jnp.concatenate(
            [y0, y1, jnp.zeros((BN, 8), jnp.float32)], axis=1)
        inv_ref[...] = inv

    big = pl.BlockSpec((BN, L), lambda i: (i, 0))
    return pl.pallas_call(
        body,
        grid=(n // BN,),
        in_specs=[
            pl.BlockSpec((NC, BN, L), lambda i: (0, i, 0)),
            big,
            pl.BlockSpec((5, 16), lambda i: (0, 0)),
            pl.BlockSpec((5, 16), lambda i: (0, 0)),
            pl.BlockSpec((5, 16), lambda i: (0, 0)),
            pl.BlockSpec((1, 16), lambda i: (0, 0)),
            pl.BlockSpec((16, 4), lambda i: (0, 0)),
            pl.BlockSpec((16, 4), lambda i: (0, 0)),
        ],
        out_specs=[big, big, pl.BlockSpec((BN, 1), lambda i: (i, 0))],
        out_shape=[
            jax.ShapeDtypeStruct((n, L), jnp.float32),
            jax.ShapeDtypeStruct((n, L), jnp.float32),
            jax.ShapeDtypeStruct((n, 1), jnp.float32),
        ],
    )(part, xdup, W10, W11, root1, b1, W20, W21)


def _tc_node2(part, h, inv, root2, b2):
    n = h.shape[0]
    BN = 2000
    assert n % BN == 0

    def body(p_ref, h_ref, inv_ref, r2, bb2, o_ref):
        acc = p_ref[0] + p_ref[1]
        agg = (acc[:, 0:4] + acc[:, 4:8]) * inv_ref[...]
        o = agg + jnp.dot(h_ref[...], r2[...],
                          preferred_element_type=jnp.float32) + bb2[...]
        m = jnp.max(o, axis=1, keepdims=True)
        s = o - m
        lse = jnp.log(jnp.sum(jnp.exp(s), axis=1, keepdims=True))
        o_ref[...] = s - lse

    big = pl.BlockSpec((BN, L), lambda i: (i, 0))
    return pl.pallas_call(
        body,
        grid=(n // BN,),
        in_specs=[
            pl.BlockSpec((NC, BN, L), lambda i: (0, i, 0)),
            big,
            pl.BlockSpec((BN, 1), lambda i: (i, 0)),
            pl.BlockSpec((16, 4), lambda i: (0, 0)),
            pl.BlockSpec((1, 4), lambda i: (0, 0)),
        ],
        out_specs=pl.BlockSpec((BN, 4), lambda i: (i, 0)),
        out_shape=jax.ShapeDtypeStruct((n, 4), jnp.float32),
    )(part, h, inv, root2, b2)


def kernel(node_feature, edge_index, edge_feature, W1, root1, bias1,
           W2, root2, bias2):
    n = node_feature.shape[0]
    E = edge_index.shape[1]

    ub = lax.bitcast_convert_type(edge_feature.reshape(1, E), jnp.int32)
    packed = jnp.concatenate([edge_index, ub], axis=0)

    xdup = jnp.concatenate(
        [node_feature, node_feature,
         jnp.ones((n, 1), jnp.float32),
         jnp.zeros((n, L - 11), jnp.float32)], axis=1)

    part1 = _sc_edge_pass(xdup, packed, n, mode=1)
    h, z, inv = _tc_node1(part1, xdup,
                          W1[0], W1[1], root1,
                          bias1.reshape(1, 16), W2[0], W2[1])
    part2 = _sc_edge_pass(z, packed, n, mode=2)
    return _tc_node2(part2, h, inv, root2, bias2.reshape(1, 4))


# speedup vs baseline: 1.7486x; 1.6469x over previous
import functools

import jax
import jax.numpy as jnp
from jax import lax
from jax.experimental import pallas as pl
from jax.experimental.pallas import tpu as pltpu
from jax.experimental.pallas import tpu_sc as plsc

NC = 2
NS = 16
L = 16
NW = NC * NS
CH = 80
K = 10
RB = 4

_GDN = lax.GatherDimensionNumbers(
    offset_dims=(), collapsed_slice_dims=(0,), start_index_map=(0,))


def _lane_gather(v, idx):
    return lax.gather(v, idx[:, None], _GDN, (1,),
                      mode=lax.GatherScatterMode.PROMISE_IN_BOUNDS)


def _sc_edge_pass(table, ei3, u2, n_nodes, mode):
    E = ei3.shape[1] * CH
    assert E % (NW * CH) == 0
    n_pad = ((n_nodes + NS * 128 - 1) // (NS * 128)) * (NS * 128)
    ept = E // NW
    nchunk = ept // CH
    ngroups = nchunk // K
    assert nchunk % K == 0 and ngroups >= 3 and (ngroups - 1) % 2 == 0
    rpt = n_pad // NS
    ZB = 128
    assert rpt % ZB == 0

    mesh = plsc.VectorSubcoreMesh(core_axis_name="c", subcore_axis_name="s")

    @functools.partial(
        pl.kernel,
        out_type=jax.ShapeDtypeStruct((NC, n_pad, L), jnp.float32),
        mesh=mesh,
        scratch_types=[
            pltpu.VMEM((2 * K, CH), jnp.int32),
            pltpu.VMEM((2 * K, CH), jnp.int32),
            pltpu.VMEM((2 * K, CH), jnp.float32),
            pltpu.VMEM((RB, CH, L), jnp.float32),
            pltpu.VMEM((RB, CH, L), jnp.float32),
            pltpu.VMEM((ZB, L), jnp.float32),
            pltpu.VMEM_SHARED((n_pad, L), jnp.float32),
            pltpu.SemaphoreType.DMA((2,)),
            pltpu.SemaphoreType.DMA((RB,)),
            pltpu.SemaphoreType.DMA((RB,)),
        ],
        compiler_params=pltpu.CompilerParams(use_tc_tiling_on_sc=False),
    )
    def kfn(table_h, ei_h, u_h, out_h, srcb, dstb, ub, rows, outr,
            zb, acc, semI, semG, semS):
        cid = lax.axis_index("c")
        sid = lax.axis_index("s")
        wid = cid * NS + sid
        base_row = sid * rpt

        lane = lax.iota(jnp.int32, L)
        if mode == 1:
            maskf = jnp.where((lane >= 5) & (lane < 10), 1.0, 0.0)
        else:
            maskf = jnp.where(lane < 4, 1.0, 0.0)

        def zrow(i, _):
            zb[i, :] = jnp.zeros((L,), jnp.float32)
            return 0
        lax.fori_loop(0, ZB, zrow, 0)

        def zcp(k, _):
            pltpu.sync_copy(zb, acc.at[pl.ds(base_row + k * ZB, ZB)])
            return 0
        lax.fori_loop(0, rpt // ZB, zcp, 0)
        plsc.subcore_barrier()

        cbase = wid * nchunk

        def issue_group(g, gb):
            g0 = cbase + g * K
            s = pl.ds(gb * K, K)
            pltpu.async_copy(ei_h.at[0, pl.ds(g0, K)], srcb.at[s],
                             semI.at[gb])
            pltpu.async_copy(ei_h.at[1, pl.ds(g0, K)], dstb.at[s],
                             semI.at[gb])
            pltpu.async_copy(u_h.at[pl.ds(g0, K)], ub.at[s], semI.at[gb])

        def wait_group(gb):
            s = pl.ds(gb * K, K)
            pltpu.make_async_copy(ei_h.at[0, pl.ds(0, K)], srcb.at[s],
                                  semI.at[gb]).wait()
            pltpu.make_async_copy(ei_h.at[1, pl.ds(0, K)], dstb.at[s],
                                  semI.at[gb]).wait()
            pltpu.make_async_copy(u_h.at[pl.ds(0, K)], ub.at[s],
                                  semI.at[gb]).wait()

        def issue_gather(row, buf):
            pltpu.async_copy(table_h.at[srcb.at[row]], rows.at[buf],
                             semG.at[buf])

        def wait_gather(buf):
            pltpu.make_async_copy(table_h.at[srcb.at[0]], rows.at[buf],
                                  semG.at[buf]).wait()

        def compute(buf, urow):
            def sub(j, _):
                u16 = jnp.clip(ub[urow, pl.ds(j * L, L)], 0.0, 1.0)
                if mode == 1:
                    a16 = u16 - 1.0
                else:
                    a16 = 1.0 - 2.0 * u16
                for i in range(L):
                    idx = jnp.full((L,), i, jnp.int32)
                    av = _lane_gather(a16, idx)
                    if mode == 1:
                        scale = maskf * av + 1.0
                    else:
                        bv = _lane_gather(u16, idx)
                        scale = maskf * av + bv
                    e = j * L + i
                    outr[buf, e, :] = rows[buf, e, :] * scale
                return 0
            lax.fori_loop(0, CH // L, sub, 0)

        def issue_scatter(buf, drow):
            pltpu.async_copy(outr.at[buf], acc.at[dstb.at[drow]],
                             semS.at[buf], add=True)

        def wait_scatter(buf):
            pltpu.make_async_copy(outr.at[buf], acc.at[dstb.at[0]],
                                  semS.at[buf]).wait()

        def do_chunk(g, gb, k, last_group):
            buf = (2 * gb + k) % RB
            if isinstance(g, int):
                if g * K + k >= RB:
                    wait_scatter(buf)
            else:
                @pl.when(g * K + k >= RB)
                def _():
                    wait_scatter(buf)
            if k == 3 and not last_group:
                issue_group(g + 1, 1 - gb)
            if k == 8 and not last_group:
                wait_group(1 - gb)
            if k <= K - 3:
                issue_gather(gb * K + k + 2, (2 * gb + k + 2) % RB)
            elif not last_group:
                k2 = k + 2 - K
                issue_gather((1 - gb) * K + k2,
                             (2 * (1 - gb) + k2) % RB)
            wait_gather(buf)
            compute(buf, gb * K + k)
            issue_scatter(buf, gb * K + k)

        issue_group(0, 0)
        wait_group(0)
        issue_gather(0, 0)
        issue_gather(1, 1)

        def main_body(gg, _):
            for gb in (0, 1):
                g = gg * 2 + gb
                for k in range(K):
                    do_chunk(g, gb, k, False)
            return 0
        lax.fori_loop(0, (ngroups - 1) // 2, main_body, 0)

        for k in range(K):
            do_chunk(ngroups - 1, 0, k, True)
        for buf in (2, 3, 0, 1):
            wait_scatter(buf)
        plsc.subcore_barrier()

        pltpu.sync_copy(acc.at[pl.ds(base_row, rpt)],
                        out_h.at[cid, pl.ds(base_row, rpt)])

    return kfn(table, ei3, u2)


def _tc_node1(part, xdup, W10, W11, root1, b1, W20, W21):
    n = xdup.shape[0]
    BN = 2000
    assert n % BN == 0

    def body(p_ref, x_ref, w10, w11, r1, bb1, w20, w21,
             h_ref, z_ref, inv_ref):
        acc = p_ref[0] + p_ref[1]
        T = acc[:, 0:5]
        S = acc[:, 5:10]
        cnt = acc[:, 10:11]
        inv = 1.0 / jnp.maximum(cnt, 1.0)
        agg = (jnp.dot(T - S, w10[...], preferred_element_type=jnp.float32)
               + jnp.dot(S, w11[...], preferred_element_type=jnp.float32))
        agg = agg * inv
        x = x_ref[:, 0:5]
        h = agg + jnp.dot(x, r1[...], preferred_element_type=jnp.float32) \
            + bb1[...]
        h = jnp.where(h > 0, h, jnp.exp(jnp.minimum(h, 0.0)) - 1.0)
        h_ref[...] = h
        y0 = jnp.dot(h, w20[...], preferred_element_type=jnp.float32)
        y1 = jnp.dot(h, w21[...], preferred_element_type=jnp.float32)
        z_ref[...] = jnp.concatenate(
            [y0, y1, jnp.zeros((BN, 8), jnp.float32)], axis=1)
        inv_ref[...] = inv

    big = pl.BlockSpec((BN, L), lambda i: (i, 0))
    return pl.pallas_call(
        body,
        grid=(n // BN,),
        in_specs=[
            pl.BlockSpec((NC, BN, L), lambda i: (0, i, 0)),
            big,
            pl.BlockSpec((5, 16), lambda i: (0, 0)),
            pl.BlockSpec((5, 16), lambda i: (0, 0)),
            pl.BlockSpec((5, 16), lambda i: (0, 0)),
            pl.BlockSpec((1, 16), lambda i: (0, 0)),
            pl.BlockSpec((16, 4), lambda i: (0, 0)),
            pl.BlockSpec((16, 4), lambda i: (0, 0)),
        ],
        out_specs=[big, big, pl.BlockSpec((BN, 1), lambda i: (i, 0))],
        out_shape=[
            jax.ShapeDtypeStruct((n, L), jnp.float32),
            jax.ShapeDtypeStruct((n, L), jnp.float32),
            jax.ShapeDtypeStruct((n, 1), jnp.float32),
        ],
    )(part, xdup, W10, W11, root1, b1, W20, W21)


def _tc_node2(part, h, inv, root2, b2):
    n = h.shape[0]
    BN = 2000
    assert n % BN == 0

    def body(p_ref, h_ref, inv_ref, r2, bb2, o_ref):
        acc = p_ref[0] + p_ref[1]
        agg = (acc[:, 0:4] + acc[:, 4:8]) * inv_ref[...]
        o = agg + jnp.dot(h_ref[...], r2[...],
                          preferred_element_type=jnp.float32) + bb2[...]
        m = jnp.max(o, axis=1, keepdims=True)
        s = o - m
        lse = jnp.log(jnp.sum(jnp.exp(s), axis=1, keepdims=True))
        o_ref[...] = s - lse

    big = pl.BlockSpec((BN, L), lambda i: (i, 0))
    return pl.pallas_call(
        body,
        grid=(n // BN,),
        in_specs=[
            pl.BlockSpec((NC, BN, L), lambda i: (0, i, 0)),
            big,
            pl.BlockSpec((BN, 1), lambda i: (i, 0)),
            pl.BlockSpec((16, 4), lambda i: (0, 0)),
            pl.BlockSpec((1, 4), lambda i: (0, 0)),
        ],
        out_specs=pl.BlockSpec((BN, 4), lambda i: (i, 0)),
        out_shape=jax.ShapeDtypeStruct((n, 4), jnp.float32),
    )(part, h, inv, root2, b2)


def kernel(node_feature, edge_index, edge_feature, W1, root1, bias1,
           W2, root2, bias2):
    n = node_feature.shape[0]
    E = edge_index.shape[1]
    ei3 = edge_index.reshape(2, E // CH, CH)
    u2 = edge_feature.reshape(E // CH, CH)

    xdup = jnp.concatenate(
        [node_feature, node_feature,
         jnp.ones((n, 1), jnp.float32),
         jnp.zeros((n, L - 11), jnp.float32)], axis=1)

    part1 = _sc_edge_pass(xdup, ei3, u2, n, mode=1)
    h, z, inv = _tc_node1(part1, xdup,
                          W1[0], W1[1], root1,
                          bias1.reshape(1, 16), W2[0], W2[1])
    part2 = _sc_edge_pass(z, ei3, u2, n, mode=2)
    return _tc_node2(part2, h, inv, root2, bias2.reshape(1, 4))


# speedup vs baseline: 1.9559x; 1.1185x over previous
import functools

import jax
import jax.numpy as jnp
from jax import lax
from jax.experimental import pallas as pl
from jax.experimental.pallas import tpu as pltpu
from jax.experimental.pallas import tpu_sc as plsc

NC = 2
NS = 16
L = 16
NW = NC * NS
CH = 80
K = 10
RB = 4

_GDN = lax.GatherDimensionNumbers(
    offset_dims=(), collapsed_slice_dims=(0,), start_index_map=(0,))


def _lane_gather(v, idx):
    return lax.gather(v, idx[:, None], _GDN, (1,),
                      mode=lax.GatherScatterMode.PROMISE_IN_BOUNDS)


def _sc_edge_pass(table, ei3, u2, n_nodes, mode):
    E = ei3.shape[1] * CH
    assert E % (NW * CH) == 0
    n_pad = ((n_nodes + NS * 128 - 1) // (NS * 128)) * (NS * 128)
    ept = E // NW
    nchunk = ept // CH
    ngroups = nchunk // K
    assert nchunk % K == 0 and ngroups >= 3 and (ngroups - 1) % 2 == 0
    rpt = n_pad // NS
    ZB = 128
    assert rpt % ZB == 0

    mesh = plsc.VectorSubcoreMesh(core_axis_name="c", subcore_axis_name="s")

    @functools.partial(
        pl.kernel,
        out_type=jax.ShapeDtypeStruct((NC, n_pad, L), jnp.float32),
        mesh=mesh,
        scratch_types=[
            pltpu.VMEM((2 * K, CH), jnp.int32),
            pltpu.VMEM((2 * K, CH), jnp.int32),
            pltpu.VMEM((2 * K, CH), jnp.float32),
            pltpu.VMEM((RB, CH, L), jnp.float32),
            pltpu.VMEM((RB, CH, L), jnp.float32),
            pltpu.VMEM((ZB, L), jnp.float32),
            pltpu.VMEM_SHARED((n_pad, L), jnp.float32),
            pltpu.SemaphoreType.DMA((2,)),
            pltpu.SemaphoreType.DMA((RB,)),
            pltpu.SemaphoreType.DMA((RB,)),
        ],
        compiler_params=pltpu.CompilerParams(use_tc_tiling_on_sc=False),
    )
    def kfn(table_h, ei_h, u_h, out_h, srcb, dstb, ub, rows, outr,
            zb, acc, semI, semG, semS):
        cid = lax.axis_index("c")
        sid = lax.axis_index("s")
        wid = cid * NS + sid
        base_row = sid * rpt

        lane = lax.iota(jnp.int32, L)
        if mode == 1:
            maskf = jnp.where((lane >= 5) & (lane < 10), 1.0, 0.0)
        else:
            maskf = jnp.where(lane < 4, 1.0, 0.0)

        def zrow(i, _):
            zb[i, :] = jnp.zeros((L,), jnp.float32)
            return 0
        lax.fori_loop(0, ZB, zrow, 0)

        def zcp(k, _):
            pltpu.sync_copy(zb, acc.at[pl.ds(base_row + k * ZB, ZB)])
            return 0
        lax.fori_loop(0, rpt // ZB, zcp, 0)
        plsc.subcore_barrier()

        cbase = wid * nchunk

        def issue_group(g, gb):
            g0 = cbase + g * K
            s = pl.ds(gb * K, K)
            pltpu.async_copy(ei_h.at[0, pl.ds(g0, K)], srcb.at[s],
                             semI.at[gb])
            pltpu.async_copy(ei_h.at[1, pl.ds(g0, K)], dstb.at[s],
                             semI.at[gb])
            pltpu.async_copy(u_h.at[pl.ds(g0, K)], ub.at[s], semI.at[gb])

        def wait_group(gb):
            s = pl.ds(gb * K, K)
            pltpu.make_async_copy(ei_h.at[0, pl.ds(0, K)], srcb.at[s],
                                  semI.at[gb]).wait()
            pltpu.make_async_copy(ei_h.at[1, pl.ds(0, K)], dstb.at[s],
                                  semI.at[gb]).wait()
            pltpu.make_async_copy(u_h.at[pl.ds(0, K)], ub.at[s],
                                  semI.at[gb]).wait()

        def issue_gather(row, buf):
            pltpu.async_copy(table_h.at[srcb.at[row]], rows.at[buf],
                             semG.at[buf])

        def wait_gather(buf):
            pltpu.make_async_copy(table_h.at[srcb.at[0]], rows.at[buf],
                                  semG.at[buf]).wait()

        def compute(buf, urow):
            def sub(j, _):
                u16 = jnp.clip(ub[urow, pl.ds(j * L, L)], 0.0, 1.0)
                if mode == 1:
                    a16 = u16 - 1.0
                else:
                    a16 = 1.0 - 2.0 * u16
                for i in range(L):
                    idx = jnp.full((L,), i, jnp.int32)
                    av = _lane_gather(a16, idx)
                    if mode == 1:
                        scale = maskf * av + 1.0
                    else:
                        bv = _lane_gather(u16, idx)
                        scale = maskf * av + bv
                    e = j * L + i
                    outr[buf, e, :] = rows[buf, e, :] * scale
                return 0
            lax.fori_loop(0, CH // L, sub, 0)

        def issue_scatter(buf, drow):
            pltpu.async_copy(outr.at[buf], acc.at[dstb.at[drow]],
                             semS.at[buf], add=True)

        def wait_scatter(buf):
            pltpu.make_async_copy(outr.at[buf], acc.at[dstb.at[0]],
                                  semS.at[buf]).wait()

        def do_chunk(g, gb, k, last_group):
            buf = (2 * gb + k) % RB
            if isinstance(g, int):
                if g * K + k >= RB:
                    wait_scatter(buf)
            else:
                @pl.when(g * K + k >= RB)
                def _():
                    wait_scatter(buf)
            if k == 3 and not last_group:
                issue_group(g + 1, 1 - gb)
            if k == 7 and not last_group:
                wait_group(1 - gb)
            if k <= K - 4:
                issue_gather(gb * K + k + 3, (2 * gb + k + 3) % RB)
            elif not last_group:
                k2 = k + 3 - K
                issue_gather((1 - gb) * K + k2,
                             (2 * (1 - gb) + k2) % RB)
            wait_gather(buf)
            compute(buf, gb * K + k)
            issue_scatter(buf, gb * K + k)

        issue_group(0, 0)
        wait_group(0)
        issue_gather(0, 0)
        issue_gather(1, 1)
        issue_gather(2, 2)

        def main_body(gg, _):
            for gb in (0, 1):
                g = gg * 2 + gb
                for k in range(K):
                    do_chunk(g, gb, k, False)
            return 0
        lax.fori_loop(0, (ngroups - 1) // 2, main_body, 0)

        for k in range(K):
            do_chunk(ngroups - 1, 0, k, True)
        for buf in (2, 3, 0, 1):
            wait_scatter(buf)
        plsc.subcore_barrier()

        pltpu.sync_copy(acc.at[pl.ds(base_row, rpt)],
                        out_h.at[cid, pl.ds(base_row, rpt)])

    return kfn(table, ei3, u2)


def _tc_node1(part, xdup, W10, W11, root1, b1, W20, W21):
    n = xdup.shape[0]
    BN = 2000
    assert n % BN == 0

    def body(p_ref, x_ref, w10, w11, r1, bb1, w20, w21,
             h_ref, z_ref, inv_ref):
        acc = p_ref[0] + p_ref[1]
        T = acc[:, 0:5]
        S = acc[:, 5:10]
        cnt = acc[:, 10:11]
        inv = 1.0 / jnp.maximum(cnt, 1.0)
        agg = (jnp.dot(T - S, w10[...], preferred_element_type=jnp.float32)
               + jnp.dot(S, w11[...], preferred_element_type=jnp.float32))
        agg = agg * inv
        x = x_ref[:, 0:5]
        h = agg + jnp.dot(x, r1[...], preferred_element_type=jnp.float32) \
            + bb1[...]
        h = jnp.where(h > 0, h, jnp.exp(jnp.minimum(h, 0.0)) - 1.0)
        h_ref[...] = h
        y0 = jnp.dot(h, w20[...], preferred_element_type=jnp.float32)
        y1 = jnp.dot(h, w21[...], preferred_element_type=jnp.float32)
        z_ref[...] = jnp.concatenate(
            [y0, y1, jnp.zeros((BN, 8), jnp.float32)], axis=1)
        inv_ref[...] = inv

    big = pl.BlockSpec((BN, L), lambda i: (i, 0))
    return pl.pallas_call(
        body,
        grid=(n // BN,),
        in_specs=[
            pl.BlockSpec((NC, BN, L), lambda i: (0, i, 0)),
            big,
            pl.BlockSpec((5, 16), lambda i: (0, 0)),
            pl.BlockSpec((5, 16), lambda i: (0, 0)),
            pl.BlockSpec((5, 16), lambda i: (0, 0)),
            pl.BlockSpec((1, 16), lambda i: (0, 0)),
            pl.BlockSpec((16, 4), lambda i: (0, 0)),
            pl.BlockSpec((16, 4), lambda i: (0, 0)),
        ],
        out_specs=[big, big, pl.BlockSpec((BN, 1), lambda i: (i, 0))],
        out_shape=[
            jax.ShapeDtypeStruct((n, L), jnp.float32),
            jax.ShapeDtypeStruct((n, L), jnp.float32),
            jax.ShapeDtypeStruct((n, 1), jnp.float32),
        ],
    )(part, xdup, W10, W11, root1, b1, W20, W21)


def _tc_node2(part, h, inv, root2, b2):
    n = h.shape[0]
    BN = 2000
    assert n % BN == 0

    def body(p_ref, h_ref, inv_ref, r2, bb2, o_ref):
        acc = p_ref[0] + p_ref[1]
        agg = (acc[:, 0:4] + acc[:, 4:8]) * inv_ref[...]
        o = agg + jnp.dot(h_ref[...], r2[...],
                          preferred_element_type=jnp.float32) + bb2[...]
        m = jnp.max(o, axis=1, keepdims=True)
        s = o - m
        lse = jnp.log(jnp.sum(jnp.exp(s), axis=1, keepdims=True))
        o_ref[...] = s - lse

    big = pl.BlockSpec((BN, L), lambda i: (i, 0))
    return pl.pallas_call(
        body,
        grid=(n // BN,),
        in_specs=[
            pl.BlockSpec((NC, BN, L), lambda i: (0, i, 0)),
            big,
            pl.BlockSpec((BN, 1), lambda i: (i, 0)),
            pl.BlockSpec((16, 4), lambda i: (0, 0)),
            pl.BlockSpec((1, 4), lambda i: (0, 0)),
        ],
        out_specs=pl.BlockSpec((BN, 4), lambda i: (i, 0)),
        out_shape=jax.ShapeDtypeStruct((n, 4), jnp.float32),
    )(part, h, inv, root2, b2)


def kernel(node_feature, edge_index, edge_feature, W1, root1, bias1,
           W2, root2, bias2):
    n = node_feature.shape[0]
    E = edge_index.shape[1]
    ei3 = edge_index.reshape(2, E // CH, CH)
    u2 = edge_feature.reshape(E // CH, CH)

    xdup = jnp.concatenate(
        [node_feature, node_feature,
         jnp.ones((n, 1), jnp.float32),
         jnp.zeros((n, L - 11), jnp.float32)], axis=1)

    part1 = _sc_edge_pass(xdup, ei3, u2, n, mode=1)
    h, z, inv = _tc_node1(part1, xdup,
                          W1[0], W1[1], root1,
                          bias1.reshape(1, 16), W2[0], W2[1])
    part2 = _sc_edge_pass(z, ei3, u2, n, mode=2)
    return _tc_node2(part2, h, inv, root2, bias2.reshape(1, 4))
